# baseline (speedup 1.0000x reference)
import functools

import jax
import jax.numpy as jnp
import numpy as np
from jax import lax
from jax.experimental import pallas as pl
from jax.experimental.pallas import tpu as pltpu

try:
    from jax.experimental.pallas import tpu_sc as plsc
    _HAS_SC = True
except ImportError:
    plsc = None
    _HAS_SC = False

H = 192
NH = 4
DK = 48
TDIM = 12
ND = 256
EC = 256
BN = 256
GC = 128
F32 = jnp.float32
I32 = jnp.int32

NODE_TYPES = ('bill_version', 'legislator_term', 'bill', 'legislator', 'donor')
N_NODES = {'bill_version': 60000, 'legislator_term': 10000, 'bill': 20000,
           'legislator': 2000, 'donor': 50000}
RELS = (('bill_version', 'is_version', 'bill'),
        ('legislator', 'samePerson', 'legislator_term'),
        ('legislator_term', 'voted_on', 'bill_version'),
        ('donor', 'donated', 'legislator_term'))
REL_E = {'is_version': 60000, 'samePerson': 10000, 'voted_on': 100000,
         'donated': 100000}


def _rup(x, m):
    return (x + m - 1) // m * m


NPAD = {nt: _rup(n, BN) for nt, n in N_NODES.items()}
NCHUNK = {}
for (_s, _r, _d) in RELS:
    _nb = NPAD[_d] // ND
    NCHUNK[_r] = _rup(_nb + -(-REL_E[_r] // EC), 16)


_GATHER_CACHE = {}


def _make_sc_gather(S, W):
    info = plsc.get_sparse_core_info()
    NC, NS = info.num_cores, info.num_subcores
    NW = NC * NS
    spw = S // NW
    iters = spw // GC
    assert spw % GC == 0 and S % NW == 0
    mesh = plsc.VectorSubcoreMesh(core_axis_name="c", subcore_axis_name="s")

    @functools.partial(
        pl.kernel, mesh=mesh,
        out_type=jax.ShapeDtypeStruct((S, W), F32),
        scratch_types=[
            pltpu.VMEM((GC,), I32),
            pltpu.VMEM((GC, W), F32),
            pltpu.SemaphoreType.DMA,
        ],
    )
    def k(tbl_hbm, idx_hbm, out_hbm, idx_v, rows_v, sem):
        wid = lax.axis_index("s") * NC + lax.axis_index("c")
        base = wid * spw

        def body(i, carry):
            off = base + i * GC
            pltpu.sync_copy(idx_hbm.at[pl.ds(off, GC)], idx_v)
            pltpu.async_copy(tbl_hbm.at[idx_v], rows_v, sem).wait()
            pltpu.sync_copy(rows_v, out_hbm.at[pl.ds(off, GC)])
            return carry

        lax.fori_loop(0, iters, body, 0)

    return k


def _gather_rows(table, idx):
    S = idx.shape[0]
    W = table.shape[1]
    key = (S, W)
    if key not in _GATHER_CACHE:
        _GATHER_CACHE[key] = _make_sc_gather(S, W)
    return _GATHER_CACHE[key](table, idx)


def _slot_plan(src, dst, n_dst_pad, nchunk):
    E = src.shape[0]
    NB = n_dst_pad // ND
    order = jnp.argsort(dst)
    ds = dst[order].astype(I32)
    ss = src[order].astype(I32)
    bid = ds // ND
    cnt = jnp.zeros((NB,), I32).at[bid].add(1)
    nch = jnp.maximum(-(-cnt // EC), 1)
    z1 = jnp.zeros((1,), I32)
    cstart = jnp.concatenate([z1, jnp.cumsum(nch).astype(I32)])
    ecum = jnp.concatenate([z1, jnp.cumsum(cnt).astype(I32)])
    ar = jnp.arange(nchunk, dtype=I32)
    blk = jnp.clip(jnp.searchsorted(cstart, ar, side='right') - 1,
                   0, NB - 1).astype(I32)
    prev = jnp.concatenate([jnp.full((1,), -1, I32), blk[:-1]])
    nxt = jnp.concatenate([blk[1:], jnp.full((1,), -2, I32)])
    fi = (blk != prev).astype(I32)
    la = (blk != nxt).astype(I32)
    ci = ar - cstart[blk]
    p0 = ecum[blk] + ci * EC
    P = p0[:, None] + jnp.arange(EC, dtype=I32)[None, :]
    valid = P < ecum[blk + 1][:, None]
    Pe = jnp.clip(P, 0, E - 1)
    dloc = jnp.where(valid, ds[Pe] - blk[:, None] * ND, ND).astype(I32)
    ssrc = jnp.where(valid, ss[Pe], 0).astype(I32).reshape(-1)
    oidx = jnp.where(valid, order[Pe].astype(I32), 0).reshape(-1)
    return {'blk': blk, 'fi': fi, 'la': la,
            'dloc': dloc.reshape(nchunk, 1, EC), 'ssrc': ssrc, 'oidx': oidx}


def _ln_in(x, eps=1e-5):
    mu = jnp.mean(x, -1, keepdims=True)
    var = jnp.mean((x - mu) ** 2, -1, keepdims=True)
    return (x - mu) * lax.rsqrt(var + eps)


def _gelu_in(x):
    return 0.5 * x * (1.0 + lax.erf(x * np.float32(1.0 / np.sqrt(2.0))))


def _headmat():
    r = lax.broadcasted_iota(I32, (H, NH), 0) // DK
    c = lax.broadcasted_iota(I32, (H, NH), 1)
    return (r == c).astype(F32)


def _dot(a, b, dims):
    return lax.dot_general(a, b, (dims, ((), ())),
                           preferred_element_type=F32)


def _proj_plain(x, ln_s, ln_b, W):
    N, din = x.shape

    def body(x_ref, s_ref, b_ref, w_ref, o_ref):
        y = _ln_in(x_ref[...]) * s_ref[0:1, :] + b_ref[0:1, :]
        o_ref[...] = _gelu_in(_dot(y, w_ref[...], ((1,), (1,))))

    return pl.pallas_call(
        body,
        grid=(N // BN,),
        in_specs=[pl.BlockSpec((BN, din), lambda i: (i, 0)),
                  pl.BlockSpec((1, din), lambda i: (0, 0)),
                  pl.BlockSpec((1, din), lambda i: (0, 0)),
                  pl.BlockSpec((H, din), lambda i: (0, 0))],
        out_specs=pl.BlockSpec((BN, H), lambda i: (i, 0)),
        out_shape=jax.ShapeDtypeStruct((N, H), F32),
    )(x, ln_s.reshape(1, din), ln_b.reshape(1, din), W)


def _proj_ts(x, t8, tp, ln_s, ln_b, W):
    N, dx = x.shape
    din = dx + TDIM

    def body(x_ref, t_ref, p_ref, s_ref, b_ref, w_ref, o_ref):
        t = t_ref[:, 0:1]
        w0 = p_ref[0, 0]
        b0 = p_ref[0, 1]
        v0 = w0 * t + b0
        wv = p_ref[1:2, 0:TDIM - 1]
        bv = p_ref[2:3, 0:TDIM - 1]
        v = jnp.sin(t * wv + bv)
        xn = jnp.concatenate([x_ref[...], v0, v], axis=1)
        y = _ln_in(xn) * s_ref[0:1, :] + b_ref[0:1, :]
        o_ref[...] = _gelu_in(_dot(y, w_ref[...], ((1,), (1,))))

    return pl.pallas_call(
        body,
        grid=(N // BN,),
        in_specs=[pl.BlockSpec((BN, dx), lambda i: (i, 0)),
                  pl.BlockSpec((BN, 8), lambda i: (i, 0)),
                  pl.BlockSpec((8, 16), lambda i: (0, 0)),
                  pl.BlockSpec((1, din), lambda i: (0, 0)),
                  pl.BlockSpec((1, din), lambda i: (0, 0)),
                  pl.BlockSpec((H, din), lambda i: (0, 0))],
        out_specs=pl.BlockSpec((BN, H), lambda i: (i, 0)),
        out_shape=jax.ShapeDtypeStruct((N, H), F32),
    )(x, t8, tp, ln_s.reshape(1, din), ln_b.reshape(1, din), W)


def _matmul(x, W):
    N = x.shape[0]
    dout = W.shape[0]

    def body(x_ref, w_ref, o_ref):
        o_ref[...] = _dot(x_ref[...], w_ref[...], ((1,), (1,)))

    return pl.pallas_call(
        body,
        grid=(N // BN,),
        in_specs=[pl.BlockSpec((BN, H), lambda i: (i, 0)),
                  pl.BlockSpec((dout, H), lambda i: (0, 0))],
        out_specs=pl.BlockSpec((BN, dout), lambda i: (i, 0)),
        out_shape=jax.ShapeDtypeStruct((N, dout), F32),
    )(x, W)


def _ffn(x, msgs, w1, w2, ls, lb):
    N = x.shape[0]
    nm = len(msgs)

    def body(*refs):
        x_ref = refs[0]
        m_refs = refs[1:1 + nm]
        w1_ref, w2_ref, s_ref, b_ref, o_ref = refs[1 + nm:]
        h = x_ref[...]
        for mr in m_refs:
            h = h + mr[...]
        y = _ln_in(_ln_in(h)) * s_ref[0:1, :] + b_ref[0:1, :]
        g = _gelu_in(_dot(y, w1_ref[...], ((1,), (1,))))
        o_ref[...] = h + _dot(g, w2_ref[...], ((1,), (1,)))

    specs = ([pl.BlockSpec((BN, H), lambda i: (i, 0))] * (1 + nm) +
             [pl.BlockSpec((4 * H, H), lambda i: (0, 0)),
              pl.BlockSpec((H, 4 * H), lambda i: (0, 0)),
              pl.BlockSpec((1, H), lambda i: (0, 0)),
              pl.BlockSpec((1, H), lambda i: (0, 0))])
    return pl.pallas_call(
        body,
        grid=(N // BN,),
        in_specs=specs,
        out_specs=pl.BlockSpec((BN, H), lambda i: (i, 0)),
        out_shape=jax.ShapeDtypeStruct((N, H), F32),
    )(x, *msgs, w1, w2, ls.reshape(1, H), lb.reshape(1, H))


def _attention(qh, kvslot, plan, R8, n_dst_pad, nchunk):
    NB = n_dst_pad // ND
    inv_sqrt_dk = np.float32(1.0 / np.sqrt(DK))

    def body(blk_ref, fi_ref, la_ref, q_ref, kv_ref, dl_ref, r_ref,
             o_ref, m_s, s_s, acc):
        j = pl.program_id(0)

        @pl.when(fi_ref[j] == 1)
        def _():
            m_s[...] = jnp.full((ND, NH), -1e30, F32)
            s_s[...] = jnp.zeros((ND, NH), F32)
            acc[...] = jnp.zeros((ND, H), F32)

        B = _headmat()
        q = q_ref[...]
        kv = kv_ref[...]
        k = kv[:, :H]
        v = kv[:, H:]
        kr = k + r_ref[0:1, :]
        dl = dl_ref[0, 0, :]
        nid = lax.broadcasted_iota(I32, (ND, EC), 0)
        ohb = nid == jnp.broadcast_to(dl[None, :], (ND, EC))
        ohf = ohb.astype(F32)
        qe = _dot(ohf, q, ((0,), (0,)))
        logit = _dot(qe * kr, B, ((1,), (0,))) * inv_sqrt_dk
        logit = jnp.where(logit >= 0, logit, 0.01 * logit)
        cmaxs = []
        for hh in range(NH):
            lb = jnp.broadcast_to(logit[:, hh][None, :], (ND, EC))
            cmaxs.append(jnp.max(jnp.where(ohb, lb, -1e30), axis=1,
                                 keepdims=True))
        cmax = jnp.concatenate(cmaxs, axis=1)
        m_old = m_s[...]
        m_new = jnp.maximum(m_old, cmax)
        scale = jnp.exp(m_old - m_new)
        me = _dot(ohf, m_new, ((0,), (0,)))
        ex = jnp.exp(logit - me)
        s_new = s_s[...] * scale + _dot(ohf, ex, ((1,), (0,)))
        w192 = _dot(ex, B, ((1,), (1,)))
        scale192 = _dot(scale, B, ((1,), (1,)))
        acc_new = acc[...] * scale192 + _dot(ohf, v * w192, ((1,), (0,)))
        m_s[...] = m_new
        s_s[...] = s_new
        acc[...] = acc_new

        @pl.when(la_ref[j] == 1)
        def _():
            s192 = _dot(s_new, B, ((1,), (1,)))
            o_ref[...] = acc_new / (s192 + 1e-16)

    grid_spec = pltpu.PrefetchScalarGridSpec(
        num_scalar_prefetch=3,
        grid=(nchunk,),
        in_specs=[
            pl.BlockSpec((ND, H), lambda j, blk, fi, la: (blk[j], 0)),
            pl.BlockSpec((EC, 2 * H), lambda j, blk, fi, la: (j, 0)),
            pl.BlockSpec((1, 1, EC), lambda j, blk, fi, la: (j, 0, 0)),
            pl.BlockSpec((8, H), lambda j, blk, fi, la: (0, 0)),
        ],
        out_specs=pl.BlockSpec((ND, H), lambda j, blk, fi, la: (blk[j], 0)),
        scratch_shapes=[pltpu.VMEM((ND, NH), F32),
                        pltpu.VMEM((ND, NH), F32),
                        pltpu.VMEM((ND, H), F32)],
    )
    return pl.pallas_call(
        body,
        grid_spec=grid_spec,
        out_shape=jax.ShapeDtypeStruct((n_dst_pad, H), F32),
    )(plan['blk'], plan['fi'], plan['la'], qh, kvslot, plan['dloc'], R8)


def _vote_seg(easlot, hltslot, plan, w1, b1, w2, b2, n_dst_pad, nchunk):
    def body(blk_ref, fi_ref, la_ref, ea_ref, hl_ref, dl_ref,
             w1_ref, b1_ref, w2_ref, b2_ref, o_ref, acc):
        j = pl.program_id(0)

        @pl.when(fi_ref[j] == 1)
        def _():
            acc[...] = jnp.zeros((ND, H), F32)

        ea = ea_ref[...]
        pol = jnp.clip(ea[:, 0:1], 0.0, 1.0)
        raw = ea[:, 1:385]
        hl = hl_ref[...][:, :H]
        f = jnp.maximum(_dot(raw, w1_ref[...], ((1,), (1,)))
                        + b1_ref[0:1, :], 0.0)
        ef = (_dot(f, w2_ref[...], ((1,), (1,))) + b2_ref[0:1, :]) \
            * (pol + 0.01)
        me = hl * ef
        dl = dl_ref[0, 0, :]
        nid = lax.broadcasted_iota(I32, (ND, EC), 0)
        ohf = (nid == jnp.broadcast_to(dl[None, :], (ND, EC))).astype(F32)
        acc_new = acc[...] + _dot(ohf, me, ((1,), (0,)))
        acc[...] = acc_new

        @pl.when(la_ref[j] == 1)
        def _():
            o_ref[...] = acc_new

    grid_spec = pltpu.PrefetchScalarGridSpec(
        num_scalar_prefetch=3,
        grid=(nchunk,),
        in_specs=[
            pl.BlockSpec((EC, 512), lambda j, blk, fi, la: (j, 0)),
            pl.BlockSpec((EC, 256), lambda j, blk, fi, la: (j, 0)),
            pl.BlockSpec((1, 1, EC), lambda j, blk, fi, la: (j, 0, 0)),
            pl.BlockSpec((H, 384), lambda j, blk, fi, la: (0, 0)),
            pl.BlockSpec((1, H), lambda j, blk, fi, la: (0, 0)),
            pl.BlockSpec((H, H), lambda j, blk, fi, la: (0, 0)),
            pl.BlockSpec((1, H), lambda j, blk, fi, la: (0, 0)),
        ],
        out_specs=pl.BlockSpec((ND, H), lambda j, blk, fi, la: (blk[j], 0)),
        scratch_shapes=[pltpu.VMEM((ND, H), F32)],
    )
    return pl.pallas_call(
        body,
        grid_spec=grid_spec,
        out_shape=jax.ShapeDtypeStruct((n_dst_pad, H), F32),
    )(plan['blk'], plan['fi'], plan['la'], easlot, hltslot, plan['dloc'],
      w1, b1.reshape(1, H), w2, b2.reshape(1, H))


def _znorm(x, msg, s, b):
    N = x.shape[0]
    nm = 0 if msg is None else 1

    def body(*refs):
        x_ref = refs[0]
        h = x_ref[...]
        if nm:
            h = h + refs[1][...]
        s_ref, b_ref, o_ref = refs[1 + nm:]
        o_ref[...] = jnp.maximum(_ln_in(h) * s_ref[0:1, :] + b_ref[0:1, :],
                                 0.0)

    args = [x] + ([msg] if nm else []) + [s.reshape(1, H), b.reshape(1, H)]
    specs = ([pl.BlockSpec((BN, H), lambda i: (i, 0))] * (1 + nm) +
             [pl.BlockSpec((1, H), lambda i: (0, 0)),
              pl.BlockSpec((1, H), lambda i: (0, 0))])
    return pl.pallas_call(
        body,
        grid=(N // BN,),
        in_specs=specs,
        out_specs=pl.BlockSpec((BN, H), lambda i: (i, 0)),
        out_shape=jax.ShapeDtypeStruct((N, H), F32),
    )(*args)


def _mean_mix_topic(zbill, zslot, plan, topicW64, n_dst_pad, nchunk):
    def body(blk_ref, fi_ref, la_ref, zb_ref, zs_ref, dl_ref, tw_ref,
             o_ref, acc, cnt):
        j = pl.program_id(0)

        @pl.when(fi_ref[j] == 1)
        def _():
            acc[...] = jnp.zeros((ND, H), F32)
            cnt[...] = jnp.zeros((ND, 8), F32)

        dl = dl_ref[0, 0, :]
        nid = lax.broadcasted_iota(I32, (ND, EC), 0)
        ohf = (nid == jnp.broadcast_to(dl[None, :], (ND, EC))).astype(F32)
        acc_new = acc[...] + _dot(ohf, zs_ref[...][:, :H], ((1,), (0,)))
        col0 = (lax.broadcasted_iota(I32, (ND, 8), 1) == 0).astype(F32)
        cnt_new = cnt[...] + col0 * jnp.sum(ohf, axis=1, keepdims=True)
        acc[...] = acc_new
        cnt[...] = cnt_new

        @pl.when(la_ref[j] == 1)
        def _():
            mean = acc_new / jnp.maximum(cnt_new[:, 0:1], 1.0)
            mix = 0.7 * zb_ref[...] + 0.3 * mean
            o_ref[...] = _dot(mix, tw_ref[...], ((1,), (1,)))

    grid_spec = pltpu.PrefetchScalarGridSpec(
        num_scalar_prefetch=3,
        grid=(nchunk,),
        in_specs=[
            pl.BlockSpec((ND, H), lambda j, blk, fi, la: (blk[j], 0)),
            pl.BlockSpec((EC, 256), lambda j, blk, fi, la: (j, 0)),
            pl.BlockSpec((1, 1, EC), lambda j, blk, fi, la: (j, 0, 0)),
            pl.BlockSpec((64, H), lambda j, blk, fi, la: (0, 0)),
        ],
        out_specs=pl.BlockSpec((ND, 64), lambda j, blk, fi, la: (blk[j], 0)),
        scratch_shapes=[pltpu.VMEM((ND, H), F32),
                        pltpu.VMEM((ND, 8), F32)],
    )
    return pl.pallas_call(
        body,
        grid_spec=grid_spec,
        out_shape=jax.ShapeDtypeStruct((n_dst_pad, 64), F32),
    )(plan['blk'], plan['fi'], plan['la'], zbill, zslot, plan['dloc'],
      topicW64)


def _pad_rows(x, n):
    return jnp.pad(x, ((0, n - x.shape[0]), (0, 0)))


def kernel(xs, ts, ea_vote, edges, params):
    p = params

    plans = {}
    for (src, rel, dst) in RELS:
        e = edges[rel]
        plans[rel] = _slot_plan(e[0], e[1], NPAD[dst], NCHUNK[rel])

    tp = jnp.zeros((8, 16), F32)
    tp = tp.at[0, 0].set(p['t2v']['w0'])
    tp = tp.at[0, 1].set(p['t2v']['b'][0])
    tp = tp.at[1, :TDIM - 1].set(p['t2v']['w'])
    tp = tp.at[2, :TDIM - 1].set(p['t2v']['b'][1:])

    h = {}
    for nt in NODE_TYPES:
        n = N_NODES[nt]
        npd = NPAD[nt]
        x = _pad_rows(xs[nt], npd)
        pp = p['proj'][nt]
        if nt in ('bill_version', 'legislator_term', 'bill'):
            t8 = jnp.broadcast_to(
                jnp.pad(ts[nt], (0, npd - n))[:, None], (npd, 8))
            h[nt] = _proj_ts(x, t8, tp, pp['ln_s'], pp['ln_b'], pp['W'])
        else:
            h[nt] = _proj_plain(x, pp['ln_s'], pp['ln_b'], pp['W'])

    for lp in p['layers']:
        Wq, Wk, Wv = lp['Q'], lp['K'], lp['V']
        Wqkv = jnp.concatenate([Wq, Wk, Wv], axis=0)
        Wkv = jnp.concatenate([Wk, Wv], axis=0)

        qkv_bv = _matmul(h['bill_version'], Wqkv)
        qkv_lt = _matmul(h['legislator_term'], Wqkv)
        q_bill = _matmul(h['bill'], Wq)
        kv_leg = _matmul(h['legislator'], Wkv)
        kv_dnr = _matmul(h['donor'], Wkv)

        qh_bv, kv_bv = qkv_bv[:, :H], qkv_bv[:, H:]
        qh_lt, kv_lt = qkv_lt[:, :H], qkv_lt[:, H:]

        kvs = {'is_version': _gather_rows(kv_bv, plans['is_version']['ssrc']),
               'voted_on': _gather_rows(kv_lt, plans['voted_on']['ssrc']),
               'samePerson': _gather_rows(kv_leg, plans['samePerson']['ssrc']),
               'donated': _gather_rows(kv_dnr, plans['donated']['ssrc'])}

        def R8(rel):
            return jnp.broadcast_to(
                lp['rel'][rel].reshape(1, H), (8, H))

        msg_bill = _attention(q_bill, kvs['is_version'], plans['is_version'],
                              R8('is_version'), NPAD['bill'],
                              NCHUNK['is_version'])
        msg_bv = _attention(qh_bv, kvs['voted_on'], plans['voted_on'],
                            R8('voted_on'), NPAD['bill_version'],
                            NCHUNK['voted_on'])
        msg_lt1 = _attention(qh_lt, kvs['samePerson'], plans['samePerson'],
                             R8('samePerson'), NPAD['legislator_term'],
                             NCHUNK['samePerson'])
        msg_lt2 = _attention(qh_lt, kvs['donated'], plans['donated'],
                             R8('donated'), NPAD['legislator_term'],
                             NCHUNK['donated'])

        w1, w2 = lp['ffn_W1'], lp['ffn_W2']
        ls, lb = lp['ffn_ln_s'], lp['ffn_ln_b']
        h = {'bill_version': _ffn(h['bill_version'], [msg_bv], w1, w2, ls, lb),
             'legislator_term': _ffn(h['legislator_term'], [msg_lt1, msg_lt2],
                                     w1, w2, ls, lb),
             'bill': _ffn(h['bill'], [msg_bill], w1, w2, ls, lb),
             'legislator': _ffn(h['legislator'], [], w1, w2, ls, lb),
             'donor': _ffn(h['donor'], [], w1, w2, ls, lb)}

    vo = plans['voted_on']
    ea_pad = jnp.pad(ea_vote, ((0, 0), (0, 512 - ea_vote.shape[1])))
    ea_slot = _gather_rows(ea_pad, vo['oidx'])
    hlt_slot = _gather_rows(
        jnp.pad(h['legislator_term'], ((0, 0), (0, 64))), vo['ssrc'])
    vw = p['vote']
    vmsg = _vote_seg(ea_slot, hlt_slot, vo, vw['W1'], vw['b1'],
                     vw['W2'], vw['b2'], NPAD['bill_version'],
                     NCHUNK['voted_on'])

    z_bv = _znorm(h['bill_version'], vmsg,
                  p['norm']['bill_version']['s'], p['norm']['bill_version']['b'])
    z_bill = _znorm(h['bill'], None,
                    p['norm']['bill']['s'], p['norm']['bill']['b'])
    iv = plans['is_version']
    zbv_slot = _gather_rows(jnp.pad(z_bv, ((0, 0), (0, 64))), iv['ssrc'])
    topicW64 = jnp.pad(p['topic_W'], ((0, 64 - p['topic_W'].shape[0]), (0, 0)))
    logits64 = _mean_mix_topic(z_bill, zbv_slot, iv, topicW64,
                               NPAD['bill'], NCHUNK['is_version'])
    return logits64[:N_NODES['bill'], :p['topic_W'].shape[0]]


# speedup vs baseline: 1.0071x; 1.0071x over previous
import functools

import jax
import jax.numpy as jnp
import numpy as np
from jax import lax
from jax.experimental import pallas as pl
from jax.experimental.pallas import tpu as pltpu

try:
    from jax.experimental.pallas import tpu_sc as plsc
    _HAS_SC = True
except ImportError:
    plsc = None
    _HAS_SC = False

H = 192
NH = 4
DK = 48
TDIM = 12
ND = 256
EC = 256
BN = 256
GC = 128
F32 = jnp.float32
I32 = jnp.int32

NODE_TYPES = ('bill_version', 'legislator_term', 'bill', 'legislator', 'donor')
N_NODES = {'bill_version': 60000, 'legislator_term': 10000, 'bill': 20000,
           'legislator': 2000, 'donor': 50000}
RELS = (('bill_version', 'is_version', 'bill'),
        ('legislator', 'samePerson', 'legislator_term'),
        ('legislator_term', 'voted_on', 'bill_version'),
        ('donor', 'donated', 'legislator_term'))
REL_E = {'is_version': 60000, 'samePerson': 10000, 'voted_on': 100000,
         'donated': 100000}


def _rup(x, m):
    return (x + m - 1) // m * m


NPAD = {nt: _rup(n, BN) for nt, n in N_NODES.items()}
NCHUNK = {}
for (_s, _r, _d) in RELS:
    _nb = NPAD[_d] // ND
    NCHUNK[_r] = _rup(_nb + -(-REL_E[_r] // EC), 16)


_GATHER_CACHE = {}


def _make_sc_gather(S, W):
    info = plsc.get_sparse_core_info()
    NC, NS = info.num_cores, info.num_subcores
    NW = NC * NS
    spw = S // NW
    gc = GC if W <= 384 else 64
    iters = spw // gc
    assert spw % gc == 0 and S % NW == 0
    pairs = iters // 2
    mesh = plsc.VectorSubcoreMesh(core_axis_name="c", subcore_axis_name="s")

    @functools.partial(
        pl.kernel, mesh=mesh,
        out_type=jax.ShapeDtypeStruct((S, W), F32),
        scratch_types=[
            pltpu.VMEM((gc,), I32), pltpu.VMEM((gc,), I32),
            pltpu.VMEM((gc, W), F32), pltpu.VMEM((gc, W), F32),
            pltpu.SemaphoreType.DMA, pltpu.SemaphoreType.DMA,
            pltpu.SemaphoreType.DMA, pltpu.SemaphoreType.DMA,
        ],
    )
    def k(tbl_hbm, idx_hbm, out_hbm, i0v, i1v, r0, r1, sg0, sg1, sw0, sw1):
        wid = lax.axis_index("s") * NC + lax.axis_index("c")
        base = wid * spw

        def start_g(iv, rv, sg, i):
            pltpu.sync_copy(idx_hbm.at[pl.ds(base + i * gc, gc)], iv)
            pltpu.async_copy(tbl_hbm.at[iv], rv, sg)

        def wait_g(iv, rv, sg):
            pltpu.make_async_copy(tbl_hbm.at[iv], rv, sg).wait()

        def start_wb(rv, sw, i):
            pltpu.async_copy(rv, out_hbm.at[pl.ds(base + i * gc, gc)], sw)

        def wait_wb(rv, sw, i):
            pltpu.make_async_copy(
                rv, out_hbm.at[pl.ds(base + i * gc, gc)], sw).wait()

        start_g(i0v, r0, sg0, 0)
        if iters > 1:
            start_g(i1v, r1, sg1, 1)

        def body(g, carry):
            i0 = 2 * g
            wait_g(i0v, r0, sg0)
            start_wb(r0, sw0, i0)
            wait_g(i1v, r1, sg1)
            start_wb(r1, sw1, i0 + 1)
            wait_wb(r0, sw0, i0)

            @pl.when(i0 + 2 < iters)
            def _():
                start_g(i0v, r0, sg0, i0 + 2)

            wait_wb(r1, sw1, i0 + 1)

            @pl.when(i0 + 3 < iters)
            def _():
                start_g(i1v, r1, sg1, i0 + 3)

            return carry

        if pairs > 0:
            lax.fori_loop(0, pairs, body, 0)
        if iters % 2 == 1:
            wait_g(i0v, r0, sg0)
            start_wb(r0, sw0, iters - 1)
            wait_wb(r0, sw0, iters - 1)

    return k


def _gather_rows(table, idx):
    S = idx.shape[0]
    W = table.shape[1]
    key = (S, W)
    if key not in _GATHER_CACHE:
        _GATHER_CACHE[key] = _make_sc_gather(S, W)
    return _GATHER_CACHE[key](table, idx)


def _slot_plan(src, dst, n_dst_pad, nchunk):
    E = src.shape[0]
    NB = n_dst_pad // ND
    order = jnp.argsort(dst)
    ds = dst[order].astype(I32)
    ss = src[order].astype(I32)
    bid = ds // ND
    cnt = jnp.zeros((NB,), I32).at[bid].add(1)
    nch = jnp.maximum(-(-cnt // EC), 1)
    z1 = jnp.zeros((1,), I32)
    cstart = jnp.concatenate([z1, jnp.cumsum(nch).astype(I32)])
    ecum = jnp.concatenate([z1, jnp.cumsum(cnt).astype(I32)])
    ar = jnp.arange(nchunk, dtype=I32)
    blk = jnp.clip(jnp.searchsorted(cstart, ar, side='right') - 1,
                   0, NB - 1).astype(I32)
    prev = jnp.concatenate([jnp.full((1,), -1, I32), blk[:-1]])
    nxt = jnp.concatenate([blk[1:], jnp.full((1,), -2, I32)])
    fi = (blk != prev).astype(I32)
    la = (blk != nxt).astype(I32)
    ci = ar - cstart[blk]
    p0 = ecum[blk] + ci * EC
    P = p0[:, None] + jnp.arange(EC, dtype=I32)[None, :]
    valid = P < ecum[blk + 1][:, None]
    Pe = jnp.clip(P, 0, E - 1)
    dloc = jnp.where(valid, ds[Pe] - blk[:, None] * ND, ND).astype(I32)
    ssrc = jnp.where(valid, ss[Pe], 0).astype(I32).reshape(-1)
    oidx = jnp.where(valid, order[Pe].astype(I32), 0).reshape(-1)
    return {'blk': blk, 'fi': fi, 'la': la,
            'dloc': dloc.reshape(nchunk, 1, EC), 'ssrc': ssrc, 'oidx': oidx}


def _ln_in(x, eps=1e-5):
    mu = jnp.mean(x, -1, keepdims=True)
    var = jnp.mean((x - mu) ** 2, -1, keepdims=True)
    return (x - mu) * lax.rsqrt(var + eps)


def _gelu_in(x):
    return 0.5 * x * (1.0 + lax.erf(x * np.float32(1.0 / np.sqrt(2.0))))


def _headmat():
    r = lax.broadcasted_iota(I32, (H, NH), 0) // DK
    c = lax.broadcasted_iota(I32, (H, NH), 1)
    return (r == c).astype(F32)


def _dot(a, b, dims):
    return lax.dot_general(a, b, (dims, ((), ())),
                           preferred_element_type=F32)


def _proj_plain(x, ln_s, ln_b, W):
    N, din = x.shape

    def body(x_ref, s_ref, b_ref, w_ref, o_ref):
        y = _ln_in(x_ref[...]) * s_ref[0:1, :] + b_ref[0:1, :]
        o_ref[...] = _gelu_in(_dot(y, w_ref[...], ((1,), (1,))))

    return pl.pallas_call(
        body,
        grid=(N // BN,),
        in_specs=[pl.BlockSpec((BN, din), lambda i: (i, 0)),
                  pl.BlockSpec((1, din), lambda i: (0, 0)),
                  pl.BlockSpec((1, din), lambda i: (0, 0)),
                  pl.BlockSpec((H, din), lambda i: (0, 0))],
        out_specs=pl.BlockSpec((BN, H), lambda i: (i, 0)),
        out_shape=jax.ShapeDtypeStruct((N, H), F32),
    )(x, ln_s.reshape(1, din), ln_b.reshape(1, din), W)


def _proj_ts(x, t8, tp, ln_s, ln_b, W):
    N, dx = x.shape
    din = dx + TDIM

    def body(x_ref, t_ref, p_ref, s_ref, b_ref, w_ref, o_ref):
        t = t_ref[:, 0:1]
        w0 = p_ref[0, 0]
        b0 = p_ref[0, 1]
        v0 = w0 * t + b0
        wv = p_ref[1:2, 0:TDIM - 1]
        bv = p_ref[2:3, 0:TDIM - 1]
        v = jnp.sin(t * wv + bv)
        xn = jnp.concatenate([x_ref[...], v0, v], axis=1)
        y = _ln_in(xn) * s_ref[0:1, :] + b_ref[0:1, :]
        o_ref[...] = _gelu_in(_dot(y, w_ref[...], ((1,), (1,))))

    return pl.pallas_call(
        body,
        grid=(N // BN,),
        in_specs=[pl.BlockSpec((BN, dx), lambda i: (i, 0)),
                  pl.BlockSpec((BN, 8), lambda i: (i, 0)),
                  pl.BlockSpec((8, 16), lambda i: (0, 0)),
                  pl.BlockSpec((1, din), lambda i: (0, 0)),
                  pl.BlockSpec((1, din), lambda i: (0, 0)),
                  pl.BlockSpec((H, din), lambda i: (0, 0))],
        out_specs=pl.BlockSpec((BN, H), lambda i: (i, 0)),
        out_shape=jax.ShapeDtypeStruct((N, H), F32),
    )(x, t8, tp, ln_s.reshape(1, din), ln_b.reshape(1, din), W)


def _matmul(x, W):
    N = x.shape[0]
    dout = W.shape[0]

    def body(x_ref, w_ref, o_ref):
        o_ref[...] = _dot(x_ref[...], w_ref[...], ((1,), (1,)))

    return pl.pallas_call(
        body,
        grid=(N // BN,),
        in_specs=[pl.BlockSpec((BN, H), lambda i: (i, 0)),
                  pl.BlockSpec((dout, H), lambda i: (0, 0))],
        out_specs=pl.BlockSpec((BN, dout), lambda i: (i, 0)),
        out_shape=jax.ShapeDtypeStruct((N, dout), F32),
    )(x, W)


def _ffn(x, msgs, w1, w2, ls, lb):
    N = x.shape[0]
    nm = len(msgs)

    def body(*refs):
        x_ref = refs[0]
        m_refs = refs[1:1 + nm]
        w1_ref, w2_ref, s_ref, b_ref, o_ref = refs[1 + nm:]
        h = x_ref[...]
        for mr in m_refs:
            h = h + mr[...]
        y = _ln_in(_ln_in(h)) * s_ref[0:1, :] + b_ref[0:1, :]
        g = _gelu_in(_dot(y, w1_ref[...], ((1,), (1,))))
        o_ref[...] = h + _dot(g, w2_ref[...], ((1,), (1,)))

    specs = ([pl.BlockSpec((BN, H), lambda i: (i, 0))] * (1 + nm) +
             [pl.BlockSpec((4 * H, H), lambda i: (0, 0)),
              pl.BlockSpec((H, 4 * H), lambda i: (0, 0)),
              pl.BlockSpec((1, H), lambda i: (0, 0)),
              pl.BlockSpec((1, H), lambda i: (0, 0))])
    return pl.pallas_call(
        body,
        grid=(N // BN,),
        in_specs=specs,
        out_specs=pl.BlockSpec((BN, H), lambda i: (i, 0)),
        out_shape=jax.ShapeDtypeStruct((N, H), F32),
    )(x, *msgs, w1, w2, ls.reshape(1, H), lb.reshape(1, H))


def _attention(qh, kvslot, plan, R8, n_dst_pad, nchunk):
    NB = n_dst_pad // ND
    inv_sqrt_dk = np.float32(1.0 / np.sqrt(DK))

    def body(blk_ref, fi_ref, la_ref, q_ref, kv_ref, dl_ref, r_ref,
             o_ref, m_s, s_s, acc):
        j = pl.program_id(0)

        @pl.when(fi_ref[j] == 1)
        def _():
            m_s[...] = jnp.full((ND, NH), -1e30, F32)
            s_s[...] = jnp.zeros((ND, NH), F32)
            acc[...] = jnp.zeros((ND, H), F32)

        B = _headmat()
        q = q_ref[...]
        kv = kv_ref[...]
        k = kv[:, :H]
        v = kv[:, H:]
        kr = k + r_ref[0:1, :]
        dl = dl_ref[0, 0, :]
        nid = lax.broadcasted_iota(I32, (ND, EC), 0)
        ohb = nid == jnp.broadcast_to(dl[None, :], (ND, EC))
        ohf = ohb.astype(F32)
        qe = _dot(ohf, q, ((0,), (0,)))
        logit = _dot(qe * kr, B, ((1,), (0,))) * inv_sqrt_dk
        logit = jnp.where(logit >= 0, logit, 0.01 * logit)
        cmaxs = []
        for hh in range(NH):
            lb = jnp.broadcast_to(logit[:, hh][None, :], (ND, EC))
            cmaxs.append(jnp.max(jnp.where(ohb, lb, -1e30), axis=1,
                                 keepdims=True))
        cmax = jnp.concatenate(cmaxs, axis=1)
        m_old = m_s[...]
        m_new = jnp.maximum(m_old, cmax)
        scale = jnp.exp(m_old - m_new)
        me = _dot(ohf, m_new, ((0,), (0,)))
        ex = jnp.exp(logit - me)
        s_new = s_s[...] * scale + _dot(ohf, ex, ((1,), (0,)))
        w192 = _dot(ex, B, ((1,), (1,)))
        scale192 = _dot(scale, B, ((1,), (1,)))
        acc_new = acc[...] * scale192 + _dot(ohf, v * w192, ((1,), (0,)))
        m_s[...] = m_new
        s_s[...] = s_new
        acc[...] = acc_new

        @pl.when(la_ref[j] == 1)
        def _():
            s192 = _dot(s_new, B, ((1,), (1,)))
            o_ref[...] = acc_new / (s192 + 1e-16)

    grid_spec = pltpu.PrefetchScalarGridSpec(
        num_scalar_prefetch=3,
        grid=(nchunk,),
        in_specs=[
            pl.BlockSpec((ND, H), lambda j, blk, fi, la: (blk[j], 0)),
            pl.BlockSpec((EC, 2 * H), lambda j, blk, fi, la: (j, 0)),
            pl.BlockSpec((1, 1, EC), lambda j, blk, fi, la: (j, 0, 0)),
            pl.BlockSpec((8, H), lambda j, blk, fi, la: (0, 0)),
        ],
        out_specs=pl.BlockSpec((ND, H), lambda j, blk, fi, la: (blk[j], 0)),
        scratch_shapes=[pltpu.VMEM((ND, NH), F32),
                        pltpu.VMEM((ND, NH), F32),
                        pltpu.VMEM((ND, H), F32)],
    )
    return pl.pallas_call(
        body,
        grid_spec=grid_spec,
        out_shape=jax.ShapeDtypeStruct((n_dst_pad, H), F32),
    )(plan['blk'], plan['fi'], plan['la'], qh, kvslot, plan['dloc'], R8)


def _vote_mlp(ea_pad, w1, b1, w2, b2):
    N = ea_pad.shape[0]
    dea = ea_pad.shape[1]

    def body(ea_ref, w1_ref, b1_ref, w2_ref, b2_ref, o_ref):
        ea = ea_ref[...]
        pol = jnp.clip(ea[:, 0:1], 0.0, 1.0)
        raw = ea[:, 1:385]
        f = jnp.maximum(_dot(raw, w1_ref[...], ((1,), (1,)))
                        + b1_ref[0:1, :], 0.0)
        ef = (_dot(f, w2_ref[...], ((1,), (1,))) + b2_ref[0:1, :]) \
            * (pol + 0.01)
        o_ref[...] = jnp.concatenate(
            [ef, jnp.zeros((ef.shape[0], 64), F32)], axis=1)

    return pl.pallas_call(
        body,
        grid=(N // BN,),
        in_specs=[pl.BlockSpec((BN, dea), lambda i: (i, 0)),
                  pl.BlockSpec((H, 384), lambda i: (0, 0)),
                  pl.BlockSpec((1, H), lambda i: (0, 0)),
                  pl.BlockSpec((H, H), lambda i: (0, 0)),
                  pl.BlockSpec((1, H), lambda i: (0, 0))],
        out_specs=pl.BlockSpec((BN, 256), lambda i: (i, 0)),
        out_shape=jax.ShapeDtypeStruct((N, 256), F32),
    )(ea_pad, w1, b1.reshape(1, H), w2, b2.reshape(1, H))


def _vote_seg(efslot, hltslot, plan, n_dst_pad, nchunk):
    def body(blk_ref, fi_ref, la_ref, ef_ref, hl_ref, dl_ref, o_ref, acc):
        j = pl.program_id(0)

        @pl.when(fi_ref[j] == 1)
        def _():
            acc[...] = jnp.zeros((ND, H), F32)

        me = hl_ref[...][:, :H] * ef_ref[...][:, :H]
        dl = dl_ref[0, 0, :]
        nid = lax.broadcasted_iota(I32, (ND, EC), 0)
        ohf = (nid == jnp.broadcast_to(dl[None, :], (ND, EC))).astype(F32)
        acc_new = acc[...] + _dot(ohf, me, ((1,), (0,)))
        acc[...] = acc_new

        @pl.when(la_ref[j] == 1)
        def _():
            o_ref[...] = acc_new

    grid_spec = pltpu.PrefetchScalarGridSpec(
        num_scalar_prefetch=3,
        grid=(nchunk,),
        in_specs=[
            pl.BlockSpec((EC, 256), lambda j, blk, fi, la: (j, 0)),
            pl.BlockSpec((EC, 256), lambda j, blk, fi, la: (j, 0)),
            pl.BlockSpec((1, 1, EC), lambda j, blk, fi, la: (j, 0, 0)),
        ],
        out_specs=pl.BlockSpec((ND, H), lambda j, blk, fi, la: (blk[j], 0)),
        scratch_shapes=[pltpu.VMEM((ND, H), F32)],
    )
    return pl.pallas_call(
        body,
        grid_spec=grid_spec,
        out_shape=jax.ShapeDtypeStruct((n_dst_pad, H), F32),
    )(plan['blk'], plan['fi'], plan['la'], efslot, hltslot, plan['dloc'])


def _znorm(x, msg, s, b):
    N = x.shape[0]
    nm = 0 if msg is None else 1

    def body(*refs):
        x_ref = refs[0]
        h = x_ref[...]
        if nm:
            h = h + refs[1][...]
        s_ref, b_ref, o_ref = refs[1 + nm:]
        o_ref[...] = jnp.maximum(_ln_in(h) * s_ref[0:1, :] + b_ref[0:1, :],
                                 0.0)

    args = [x] + ([msg] if nm else []) + [s.reshape(1, H), b.reshape(1, H)]
    specs = ([pl.BlockSpec((BN, H), lambda i: (i, 0))] * (1 + nm) +
             [pl.BlockSpec((1, H), lambda i: (0, 0)),
              pl.BlockSpec((1, H), lambda i: (0, 0))])
    return pl.pallas_call(
        body,
        grid=(N // BN,),
        in_specs=specs,
        out_specs=pl.BlockSpec((BN, H), lambda i: (i, 0)),
        out_shape=jax.ShapeDtypeStruct((N, H), F32),
    )(*args)


def _mean_mix_topic(zbill, zslot, plan, topicW64, n_dst_pad, nchunk):
    def body(blk_ref, fi_ref, la_ref, zb_ref, zs_ref, dl_ref, tw_ref,
             o_ref, acc, cnt):
        j = pl.program_id(0)

        @pl.when(fi_ref[j] == 1)
        def _():
            acc[...] = jnp.zeros((ND, H), F32)
            cnt[...] = jnp.zeros((ND, 8), F32)

        dl = dl_ref[0, 0, :]
        nid = lax.broadcasted_iota(I32, (ND, EC), 0)
        ohf = (nid == jnp.broadcast_to(dl[None, :], (ND, EC))).astype(F32)
        acc_new = acc[...] + _dot(ohf, zs_ref[...][:, :H], ((1,), (0,)))
        col0 = (lax.broadcasted_iota(I32, (ND, 8), 1) == 0).astype(F32)
        cnt_new = cnt[...] + col0 * jnp.sum(ohf, axis=1, keepdims=True)
        acc[...] = acc_new
        cnt[...] = cnt_new

        @pl.when(la_ref[j] == 1)
        def _():
            mean = acc_new / jnp.maximum(cnt_new[:, 0:1], 1.0)
            mix = 0.7 * zb_ref[...] + 0.3 * mean
            o_ref[...] = _dot(mix, tw_ref[...], ((1,), (1,)))

    grid_spec = pltpu.PrefetchScalarGridSpec(
        num_scalar_prefetch=3,
        grid=(nchunk,),
        in_specs=[
            pl.BlockSpec((ND, H), lambda j, blk, fi, la: (blk[j], 0)),
            pl.BlockSpec((EC, 256), lambda j, blk, fi, la: (j, 0)),
            pl.BlockSpec((1, 1, EC), lambda j, blk, fi, la: (j, 0, 0)),
            pl.BlockSpec((64, H), lambda j, blk, fi, la: (0, 0)),
        ],
        out_specs=pl.BlockSpec((ND, 64), lambda j, blk, fi, la: (blk[j], 0)),
        scratch_shapes=[pltpu.VMEM((ND, H), F32),
                        pltpu.VMEM((ND, 8), F32)],
    )
    return pl.pallas_call(
        body,
        grid_spec=grid_spec,
        out_shape=jax.ShapeDtypeStruct((n_dst_pad, 64), F32),
    )(plan['blk'], plan['fi'], plan['la'], zbill, zslot, plan['dloc'],
      topicW64)


def _pad_rows(x, n):
    return jnp.pad(x, ((0, n - x.shape[0]), (0, 0)))


def kernel(xs, ts, ea_vote, edges, params):
    p = params

    plans = {}
    for (src, rel, dst) in RELS:
        e = edges[rel]
        plans[rel] = _slot_plan(e[0], e[1], NPAD[dst], NCHUNK[rel])

    tp = jnp.zeros((8, 16), F32)
    tp = tp.at[0, 0].set(p['t2v']['w0'])
    tp = tp.at[0, 1].set(p['t2v']['b'][0])
    tp = tp.at[1, :TDIM - 1].set(p['t2v']['w'])
    tp = tp.at[2, :TDIM - 1].set(p['t2v']['b'][1:])

    h = {}
    for nt in NODE_TYPES:
        n = N_NODES[nt]
        npd = NPAD[nt]
        x = _pad_rows(xs[nt], npd)
        pp = p['proj'][nt]
        if nt in ('bill_version', 'legislator_term', 'bill'):
            t8 = jnp.broadcast_to(
                jnp.pad(ts[nt], (0, npd - n))[:, None], (npd, 8))
            h[nt] = _proj_ts(x, t8, tp, pp['ln_s'], pp['ln_b'], pp['W'])
        else:
            h[nt] = _proj_plain(x, pp['ln_s'], pp['ln_b'], pp['W'])

    for lp in p['layers']:
        Wq, Wk, Wv = lp['Q'], lp['K'], lp['V']
        Wqkv = jnp.concatenate([Wq, Wk, Wv], axis=0)
        Wkv = jnp.concatenate([Wk, Wv], axis=0)

        qkv_bv = _matmul(h['bill_version'], Wqkv)
        qkv_lt = _matmul(h['legislator_term'], Wqkv)
        q_bill = _matmul(h['bill'], Wq)
        kv_leg = _matmul(h['legislator'], Wkv)
        kv_dnr = _matmul(h['donor'], Wkv)

        qh_bv, kv_bv = qkv_bv[:, :H], qkv_bv[:, H:]
        qh_lt, kv_lt = qkv_lt[:, :H], qkv_lt[:, H:]

        kvs = {'is_version': _gather_rows(kv_bv, plans['is_version']['ssrc']),
               'voted_on': _gather_rows(kv_lt, plans['voted_on']['ssrc']),
               'samePerson': _gather_rows(kv_leg, plans['samePerson']['ssrc']),
               'donated': _gather_rows(kv_dnr, plans['donated']['ssrc'])}

        def R8(rel):
            return jnp.broadcast_to(
                lp['rel'][rel].reshape(1, H), (8, H))

        msg_bill = _attention(q_bill, kvs['is_version'], plans['is_version'],
                              R8('is_version'), NPAD['bill'],
                              NCHUNK['is_version'])
        msg_bv = _attention(qh_bv, kvs['voted_on'], plans['voted_on'],
                            R8('voted_on'), NPAD['bill_version'],
                            NCHUNK['voted_on'])
        msg_lt1 = _attention(qh_lt, kvs['samePerson'], plans['samePerson'],
                             R8('samePerson'), NPAD['legislator_term'],
                             NCHUNK['samePerson'])
        msg_lt2 = _attention(qh_lt, kvs['donated'], plans['donated'],
                             R8('donated'), NPAD['legislator_term'],
                             NCHUNK['donated'])

        w1, w2 = lp['ffn_W1'], lp['ffn_W2']
        ls, lb = lp['ffn_ln_s'], lp['ffn_ln_b']
        h = {'bill_version': _ffn(h['bill_version'], [msg_bv], w1, w2, ls, lb),
             'legislator_term': _ffn(h['legislator_term'], [msg_lt1, msg_lt2],
                                     w1, w2, ls, lb),
             'bill': _ffn(h['bill'], [msg_bill], w1, w2, ls, lb),
             'legislator': _ffn(h['legislator'], [], w1, w2, ls, lb),
             'donor': _ffn(h['donor'], [], w1, w2, ls, lb)}

    vo = plans['voted_on']
    E_vo = ea_vote.shape[0]
    ea_pad = jnp.pad(ea_vote, ((0, _rup(E_vo, BN) - E_vo), (0, 0)))
    vw = p['vote']
    e_feat = _vote_mlp(ea_pad, vw['W1'], vw['b1'], vw['W2'], vw['b2'])
    ef_slot = _gather_rows(e_feat, vo['oidx'])
    hlt_slot = _gather_rows(
        jnp.pad(h['legislator_term'], ((0, 0), (0, 64))), vo['ssrc'])
    vmsg = _vote_seg(ef_slot, hlt_slot, vo, NPAD['bill_version'],
                     NCHUNK['voted_on'])

    z_bv = _znorm(h['bill_version'], vmsg,
                  p['norm']['bill_version']['s'], p['norm']['bill_version']['b'])
    z_bill = _znorm(h['bill'], None,
                    p['norm']['bill']['s'], p['norm']['bill']['b'])
    iv = plans['is_version']
    zbv_slot = _gather_rows(jnp.pad(z_bv, ((0, 0), (0, 64))), iv['ssrc'])
    topicW64 = jnp.pad(p['topic_W'], ((0, 64 - p['topic_W'].shape[0]), (0, 0)))
    logits64 = _mean_mix_topic(z_bill, zbv_slot, iv, topicW64,
                               NPAD['bill'], NCHUNK['is_version'])
    return logits64[:N_NODES['bill'], :p['topic_W'].shape[0]]
